# Initial kernel scaffold; baseline (speedup 1.0000x reference)
import functools

import jax
import jax.numpy as jnp
from jax import lax
from jax.experimental import pallas as pl
from jax.experimental.pallas import tpu as pltpu
from jax.experimental.pallas import tpu_sc as plsc

N = 10000
E = 320000
D = 128
NP = 10240
EP = 327680
NT = 32
EPT = EP // NT
K = 128
NCH = EPT // K
VPT = EPT // 16
NV = NP // 16

_mesh = plsc.VectorSubcoreMesh(core_axis_name="c", subcore_axis_name="s")


def _lane():
    return lax.broadcasted_iota(jnp.int32, (16,), 0)


def _take16(v, idx):
    return jnp.take(v, idx, mode="promise_in_bounds")


def _seg_combine(k, v, op):
    lane = _lane()
    for sh in (1, 2, 4, 8):
        idx = jnp.maximum(lane - sh, 0)
        kz = _take16(k, idx)
        vz = _take16(v, idx)
        v = jnp.where((kz == k) & (lane >= sh), op(v, vz), v)
    nxt = _take16(k, jnp.minimum(lane + 1, 15))
    last = (k != nxt) | (lane == 15)
    return v, last


def _edge_logit(s1_v, s2_v, d16, s16):
    e = plsc.load_gather(s1_v, [d16]) + plsc.load_gather(s2_v, [s16])
    return jnp.where(e >= 0, e, 0.01 * e)


def _wid():
    return lax.axis_index("c") * 16 + lax.axis_index("s")


def _combine_partials(part_h, row_v, acc_v, op):
    pltpu.sync_copy(part_h.at[0], acc_v)

    def outer(r, _):
        pltpu.sync_copy(part_h.at[r], row_v)

        def inner(j, _):
            sl = pl.ds(j * 16, 16)
            acc_v[sl] = op(acc_v[sl], row_v[sl])
            return 0

        lax.fori_loop(0, NV, inner, 0)
        return 0

    lax.fori_loop(1, NT, outer, 0)


@functools.partial(
    pl.kernel,
    out_type=jax.ShapeDtypeStruct((NT, NP), jnp.float32),
    mesh=_mesh,
    scratch_types=[
        pltpu.VMEM((NP,), jnp.float32),
        pltpu.VMEM((NP,), jnp.float32),
        pltpu.VMEM((EPT,), jnp.int32),
        pltpu.VMEM((EPT,), jnp.int32),
        pltpu.VMEM((NP,), jnp.float32),
    ],
)
def _segmax(s1_h, s2_h, src_h, dst_h, mpart_h, s1_v, s2_v, src_v, dst_v, m_v):
    w = _wid()
    pltpu.sync_copy(s1_h, s1_v)
    pltpu.sync_copy(s2_h, s2_v)
    pltpu.sync_copy(src_h.at[w], src_v)
    pltpu.sync_copy(dst_h.at[w], dst_v)

    def init(j, _):
        m_v[pl.ds(j * 16, 16)] = jnp.full((16,), -1e30, jnp.float32)
        return 0

    lax.fori_loop(0, NV, init, 0)

    def step(j, _):
        sl = pl.ds(j * 16, 16)
        d16 = dst_v[sl]
        e = _edge_logit(s1_v, s2_v, d16, src_v[sl])
        k, v = plsc.sort_key_val(d16, e)
        v, last = _seg_combine(k, v, jnp.maximum)
        old = plsc.load_gather(m_v, [k])
        plsc.store_scatter(m_v, [k], jnp.maximum(old, v), mask=last)
        return 0

    lax.fori_loop(0, VPT, step, 0)
    pltpu.sync_copy(m_v, mpart_h.at[w])


@functools.partial(
    pl.kernel,
    out_type=(
        jax.ShapeDtypeStruct((NT, NP), jnp.float32),
        jax.ShapeDtypeStruct((NT, EPT), jnp.float32),
    ),
    mesh=_mesh,
    scratch_types=[
        pltpu.VMEM((NP,), jnp.float32),
        pltpu.VMEM((NP,), jnp.float32),
        pltpu.VMEM((EPT,), jnp.int32),
        pltpu.VMEM((EPT,), jnp.int32),
        pltpu.VMEM((NP,), jnp.float32),
        pltpu.VMEM((NP,), jnp.float32),
        pltpu.VMEM((NP,), jnp.float32),
        pltpu.VMEM((EPT,), jnp.float32),
    ],
)
def _denom(s1_h, s2_h, src_h, dst_h, mpart_h, dpart_h, eexp_h,
           s1_v, s2_v, src_v, dst_v, m_v, row_v, den_v, ex_v):
    w = _wid()
    pltpu.sync_copy(s1_h, s1_v)
    pltpu.sync_copy(s2_h, s2_v)
    pltpu.sync_copy(src_h.at[w], src_v)
    pltpu.sync_copy(dst_h.at[w], dst_v)
    _combine_partials(mpart_h, row_v, m_v, jnp.maximum)

    def init(j, _):
        den_v[pl.ds(j * 16, 16)] = jnp.zeros((16,), jnp.float32)
        return 0

    lax.fori_loop(0, NV, init, 0)

    def step(j, _):
        sl = pl.ds(j * 16, 16)
        d16 = dst_v[sl]
        e = _edge_logit(s1_v, s2_v, d16, src_v[sl])
        ex = jnp.exp(e - plsc.load_gather(m_v, [d16]))
        ex_v[sl] = ex
        k, v = plsc.sort_key_val(d16, ex)
        v, last = _seg_combine(k, v, lambda x, y: x + y)
        plsc.addupdate_scatter(den_v, [k], v, mask=last)
        return 0

    lax.fori_loop(0, VPT, step, 0)
    pltpu.sync_copy(den_v, dpart_h.at[w])
    pltpu.sync_copy(ex_v, eexp_h.at[w])


@functools.partial(
    pl.kernel,
    out_type=jax.ShapeDtypeStruct((2, NP, D), jnp.float32),
    mesh=_mesh,
    scratch_types=[
        pltpu.VMEM((EPT,), jnp.int32),
        pltpu.VMEM((NCH, K), jnp.int32),
        pltpu.VMEM((EPT,), jnp.float32),
        pltpu.VMEM((NP,), jnp.float32),
        pltpu.VMEM((NP,), jnp.float32),
        pltpu.VMEM((K, D), jnp.float32),
        pltpu.VMEM((K, D), jnp.float32),
        pltpu.VMEM_SHARED((NP, D), jnp.float32),
        pltpu.SemaphoreType.DMA,
        pltpu.SemaphoreType.DMA,
    ],
)
def _aggregate(z_h, src_h, dst3_h, eexp_h, dpart_h, res_h,
               src_v, dst3_v, ex_v, row_v, inv_v, rows0, rows1, res_sh,
               sem0, sem1):
    w = _wid()
    cid = lax.axis_index("c")
    sid = lax.axis_index("s")
    pltpu.sync_copy(src_h.at[w], src_v)
    pltpu.sync_copy(dst3_h.at[w], dst3_v)
    pltpu.sync_copy(eexp_h.at[w], ex_v)
    _combine_partials(dpart_h, row_v, inv_v, lambda x, y: x + y)

    def invert(j, _):
        sl = pl.ds(j * 16, 16)
        inv_v[sl] = 1.0 / inv_v[sl]
        return 0

    lax.fori_loop(0, NV, invert, 0)

    def zrow(r, _):
        for s in range(8):
            rows0[r, pl.ds(s * 16, 16)] = jnp.zeros((16,), jnp.float32)
        return 0

    lax.fori_loop(0, K, zrow, 0)
    base = w * (NP // NT)
    pltpu.sync_copy(rows0, res_sh.at[pl.ds(base, K)])
    pltpu.sync_copy(rows0.at[pl.ds(0, NP // NT - K)],
                    res_sh.at[pl.ds(base + K, NP // NT - K)])
    plsc.subcore_barrier()

    def start_gather(c, buf, sem):
        pltpu.async_copy(z_h.at[src_v.at[pl.ds(c * K, K)]], buf, sem)

    def wait_gather(buf, sem):
        pltpu.make_async_copy(z_h.at[pl.ds(0, K)], buf, sem).wait()

    def scale_and_scatter(c, buf):
        def jbody(j, _):
            sl16 = pl.ds(j * 16, 16)
            d16 = dst3_v[c, sl16]
            w16 = ex_v[pl.ds(c * K + j * 16, 16)] * plsc.load_gather(inv_v, [d16])
            for t in range(16):
                sp = _take16(w16, jnp.full((16,), t, jnp.int32))
                r = j * 16 + t
                for s in range(8):
                    csl = pl.ds(s * 16, 16)
                    buf[r, csl] = buf[r, csl] * sp
            return 0

        lax.fori_loop(0, K // 16, jbody, 0)
        pltpu.sync_copy(buf, res_sh.at[dst3_v.at[c]], add=True)

    start_gather(0, rows0, sem0)

    def pair(p, _):
        c0 = 2 * p
        c1 = 2 * p + 1
        start_gather(c1, rows1, sem1)
        wait_gather(rows0, sem0)
        scale_and_scatter(c0, rows0)
        start_gather(jnp.minimum(c0 + 2, NCH - 2), rows0, sem0)
        wait_gather(rows1, sem1)
        scale_and_scatter(c1, rows1)
        return 0

    lax.fori_loop(0, NCH // 2, pair, 0)
    wait_gather(rows0, sem0)

    plsc.subcore_barrier()
    rows_per_tile = NP // 16
    fbase = sid * rows_per_tile
    pltpu.sync_copy(res_sh.at[pl.ds(fbase, rows_per_tile)],
                    res_h.at[cid, pl.ds(fbase, rows_per_tile)])


def _tc1_body(x_ref, w1_ref, w2_ref, ap_ref, z_ref, s_ref):
    z1 = jnp.maximum(
        lax.dot_general(x_ref[...], w1_ref[...], (((1,), (1,)), ((), ())),
                        preferred_element_type=jnp.float32), 0.0)
    z2 = jnp.maximum(
        lax.dot_general(z1, w2_ref[...], (((1,), (1,)), ((), ())),
                        preferred_element_type=jnp.float32), 0.0)
    z_ref[...] = z2
    s_ref[...] = lax.dot_general(z2, ap_ref[...], (((1,), (0,)), ((), ())),
                                 preferred_element_type=jnp.float32)


def _tc2_body(z_ref, r0_ref, r1_ref, o_ref):
    o_ref[...] = jnp.maximum(z_ref[...] - r0_ref[...] - r1_ref[...], 0.0)


_B1 = 1024


def _dense_fwd(xp, W1, W2, ap):
    return pl.pallas_call(
        _tc1_body,
        grid=(NP // _B1,),
        in_specs=[
            pl.BlockSpec((_B1, D), lambda i: (i, 0)),
            pl.BlockSpec((D, D), lambda i: (0, 0)),
            pl.BlockSpec((D, D), lambda i: (0, 0)),
            pl.BlockSpec((D, D), lambda i: (0, 0)),
        ],
        out_specs=[
            pl.BlockSpec((_B1, D), lambda i: (i, 0)),
            pl.BlockSpec((_B1, D), lambda i: (i, 0)),
        ],
        out_shape=[
            jax.ShapeDtypeStruct((NP, D), jnp.float32),
            jax.ShapeDtypeStruct((NP, D), jnp.float32),
        ],
    )(xp, W1, W2, ap)


def _final(z, r0, r1):
    return pl.pallas_call(
        _tc2_body,
        grid=(NP // _B1,),
        in_specs=[
            pl.BlockSpec((_B1, D), lambda i: (i, 0)),
            pl.BlockSpec((_B1, D), lambda i: (i, 0)),
            pl.BlockSpec((_B1, D), lambda i: (i, 0)),
        ],
        out_specs=pl.BlockSpec((_B1, D), lambda i: (i, 0)),
        out_shape=jax.ShapeDtypeStruct((NP, D), jnp.float32),
    )(z, r0, r1)


def kernel(x, edge_index, W1, W2, a):
    xp = jnp.zeros((NP, D), jnp.float32).at[:N].set(x)
    ap = jnp.zeros((D, D), jnp.float32)
    ap = ap.at[:, 0].set(a[:D, 0]).at[:, 1].set(a[D:, 0])

    pad = jnp.full((EP - E,), N, jnp.int32)
    src = jnp.concatenate([edge_index[0], pad])
    dst = jnp.concatenate([edge_index[1], pad])
    src2 = src.reshape(NT, EPT)
    dst2 = dst.reshape(NT, EPT)
    dst3 = dst.reshape(NT, NCH, K)

    z, sfull = _dense_fwd(xp, W1, W2, ap)
    s1 = sfull[:, 0]
    s2 = sfull[:, 1]

    mpart = _segmax(s1, s2, src2, dst2)
    dpart, eexp = _denom(s1, s2, src2, dst2, mpart)
    res = _aggregate(z, src2, dst3, eexp, dpart)

    out = _final(z, res[0], res[1])
    return out[:N]


# speedup vs baseline: 7.1203x; 7.1203x over previous
import functools

import jax
import jax.numpy as jnp
from jax import lax
from jax.experimental import pallas as pl
from jax.experimental.pallas import tpu as pltpu
from jax.experimental.pallas import tpu_sc as plsc

N = 10000
E = 320000
D = 128
NP = 10240
EP = 327680
NT = 32
EPT = EP // NT
K = 128
NCH = EPT // K
VPT = EPT // 16
NV = NP // 16

_mesh = plsc.VectorSubcoreMesh(core_axis_name="c", subcore_axis_name="s")
_mesh1 = plsc.VectorSubcoreMesh(core_axis_name="c", subcore_axis_name="s",
                                num_cores=1)
_sc_params = pltpu.CompilerParams(needs_layout_passes=False)
NT1 = 16
EPT1 = EP // NT1
NCH1 = EPT1 // K


def _lane():
    return lax.broadcasted_iota(jnp.int32, (16,), 0)


_GDN = lax.GatherDimensionNumbers(
    offset_dims=(), collapsed_slice_dims=(0,), start_index_map=(0,))


def _take16(v, idx):
    return lax.gather(v, idx[:, None], _GDN, (1,),
                      mode=lax.GatherScatterMode.PROMISE_IN_BOUNDS)


def _seg_combine(k, v, op):
    lane = _lane()
    for sh in (1, 2, 4, 8):
        idx = jnp.maximum(lane - sh, 0)
        kz = _take16(k, idx)
        vz = _take16(v, idx)
        v = jnp.where((kz == k) & (lane >= sh), op(v, vz), v)
    nxt = _take16(k, jnp.minimum(lane + 1, 15))
    last = (k != nxt) | (lane == 15)
    return v, last


def _edge_logit(s1_v, s2_v, d16, s16):
    e = plsc.load_gather(s1_v, [d16]) + plsc.load_gather(s2_v, [s16])
    return jnp.where(e >= 0, e, 0.01 * e)


def _wid():
    return lax.axis_index("c") * 16 + lax.axis_index("s")


def _combine_partials(part_h, row_v, acc_v, op):
    pltpu.sync_copy(part_h.at[0], acc_v)

    def outer(r, _):
        pltpu.sync_copy(part_h.at[r], row_v)

        def inner(j, _):
            sl = pl.ds(j * 16, 16)
            acc_v[sl] = op(acc_v[sl], row_v[sl])
            return 0

        lax.fori_loop(0, NV, inner, 0)
        return 0

    lax.fori_loop(1, NT, outer, 0)


@functools.partial(
    pl.kernel,
    out_type=jax.ShapeDtypeStruct((NT, NP), jnp.float32),
    mesh=_mesh,
    compiler_params=_sc_params,
    scratch_types=[
        pltpu.VMEM((NP,), jnp.float32),
        pltpu.VMEM((NP,), jnp.float32),
        pltpu.VMEM((EPT,), jnp.int32),
        pltpu.VMEM((EPT,), jnp.int32),
        pltpu.VMEM((NP,), jnp.float32),
    ],
)
def _segmax(s1_h, s2_h, src_h, dst_h, mpart_h, s1_v, s2_v, src_v, dst_v, m_v):
    w = _wid()
    pltpu.sync_copy(s1_h, s1_v)
    pltpu.sync_copy(s2_h, s2_v)
    pltpu.sync_copy(src_h.at[w], src_v)
    pltpu.sync_copy(dst_h.at[w], dst_v)

    def init(j, _):
        m_v[pl.ds(j * 16, 16)] = jnp.full((16,), -1e30, jnp.float32)
        return 0

    lax.fori_loop(0, NV, init, 0)

    def step(j, _):
        sl = pl.ds(j * 16, 16)
        d16 = dst_v[sl]
        e = _edge_logit(s1_v, s2_v, d16, src_v[sl])
        k, v = plsc.sort_key_val(d16, e)
        v, last = _seg_combine(k, v, jnp.maximum)
        old = plsc.load_gather(m_v, [k])
        plsc.store_scatter(m_v, [k], jnp.maximum(old, v), mask=last)
        return 0

    lax.fori_loop(0, VPT, step, 0)
    pltpu.sync_copy(m_v, mpart_h.at[w])


@functools.partial(
    pl.kernel,
    out_type=(
        jax.ShapeDtypeStruct((NT, NP), jnp.float32),
        jax.ShapeDtypeStruct((NT, EPT), jnp.float32),
    ),
    mesh=_mesh,
    compiler_params=_sc_params,
    scratch_types=[
        pltpu.VMEM((NP,), jnp.float32),
        pltpu.VMEM((NP,), jnp.float32),
        pltpu.VMEM((EPT,), jnp.int32),
        pltpu.VMEM((EPT,), jnp.int32),
        pltpu.VMEM((NP,), jnp.float32),
        pltpu.VMEM((NP,), jnp.float32),
        pltpu.VMEM((NP,), jnp.float32),
        pltpu.VMEM((EPT,), jnp.float32),
    ],
)
def _denom(s1_h, s2_h, src_h, dst_h, mpart_h, dpart_h, eexp_h,
           s1_v, s2_v, src_v, dst_v, m_v, row_v, den_v, ex_v):
    w = _wid()
    pltpu.sync_copy(s1_h, s1_v)
    pltpu.sync_copy(s2_h, s2_v)
    pltpu.sync_copy(src_h.at[w], src_v)
    pltpu.sync_copy(dst_h.at[w], dst_v)
    _combine_partials(mpart_h, row_v, m_v, jnp.maximum)

    def init(j, _):
        den_v[pl.ds(j * 16, 16)] = jnp.zeros((16,), jnp.float32)
        return 0

    lax.fori_loop(0, NV, init, 0)

    def step(j, _):
        sl = pl.ds(j * 16, 16)
        d16 = dst_v[sl]
        e = _edge_logit(s1_v, s2_v, d16, src_v[sl])
        ex = jnp.exp(e - plsc.load_gather(m_v, [d16]))
        ex_v[sl] = ex
        k, v = plsc.sort_key_val(d16, ex)
        v, last = _seg_combine(k, v, lambda x, y: x + y)
        plsc.addupdate_scatter(den_v, [k], v, mask=last)
        return 0

    lax.fori_loop(0, VPT, step, 0)
    pltpu.sync_copy(den_v, dpart_h.at[w])
    pltpu.sync_copy(ex_v, eexp_h.at[w])


@functools.partial(
    pl.kernel,
    out_type=jax.ShapeDtypeStruct((NT, EPT), jnp.float32),
    mesh=_mesh,
    compiler_params=_sc_params,
    scratch_types=[
        pltpu.VMEM((EPT,), jnp.int32),
        pltpu.VMEM((EPT,), jnp.float32),
        pltpu.VMEM((NP,), jnp.float32),
        pltpu.VMEM((NP,), jnp.float32),
    ],
)
def _weights(dpart_h, eexp_h, dst_h, w_h, dst_v, ex_v, row_v, inv_v):
    w = _wid()
    pltpu.sync_copy(dst_h.at[w], dst_v)
    pltpu.sync_copy(eexp_h.at[w], ex_v)
    _combine_partials(dpart_h, row_v, inv_v, lambda x, y: x + y)

    def invert(j, _):
        sl = pl.ds(j * 16, 16)
        inv_v[sl] = 1.0 / inv_v[sl]
        return 0

    lax.fori_loop(0, NV, invert, 0)

    def wconv(j, _):
        sl = pl.ds(j * 16, 16)
        ex_v[sl] = ex_v[sl] * plsc.load_gather(inv_v, [dst_v[sl]])
        return 0

    lax.fori_loop(0, VPT, wconv, 0)
    pltpu.sync_copy(ex_v, w_h.at[w])


@functools.partial(
    pl.kernel,
    out_type=jax.ShapeDtypeStruct((NP, D), jnp.float32),
    mesh=_mesh1,
    compiler_params=_sc_params,
    scratch_types=[
        pltpu.VMEM((2, K), jnp.int32),
        pltpu.VMEM((2, K), jnp.int32),
        pltpu.VMEM((2, K), jnp.float32),
        pltpu.VMEM((K, D), jnp.float32),
        pltpu.VMEM((K, D), jnp.float32),
        pltpu.VMEM_SHARED((NP, D), jnp.float32),
        pltpu.SemaphoreType.DMA,
        pltpu.SemaphoreType.DMA,
    ],
)
def _aggregate(z_h, src_h, dst_h, w_h, res_h,
               sidx, didx, wbuf, rows0, rows1, res_sh, sem0, sem1):
    sid = lax.axis_index("s")
    bufs = (rows0, rows1)
    sems = (sem0, sem1)

    def zrow(r, _):
        for s in range(8):
            rows0[r, pl.ds(s * 16, 16)] = jnp.zeros((16,), jnp.float32)
        return 0

    lax.fori_loop(0, K, zrow, 0)
    base = sid * (NP // NT1)
    for q in range(NP // NT1 // K):
        pltpu.sync_copy(rows0, res_sh.at[pl.ds(base + q * K, K)])
    plsc.subcore_barrier()

    def start(c, b):
        pltpu.sync_copy(src_h.at[sid, pl.ds(c * K, K)], sidx.at[b])
        pltpu.async_copy(z_h.at[sidx.at[b]], bufs[b], sems[b])

    def process(c, b):
        pltpu.sync_copy(dst_h.at[sid, pl.ds(c * K, K)], didx.at[b])
        pltpu.sync_copy(w_h.at[sid, pl.ds(c * K, K)], wbuf.at[b])
        pltpu.make_async_copy(z_h.at[pl.ds(0, K)], bufs[b], sems[b]).wait()
        buf = bufs[b]

        def jbody(j, _):
            w16 = wbuf[b, pl.ds(j * 16, 16)]
            for t in range(16):
                sp = _take16(w16, jnp.full((16,), t, jnp.int32))
                r = j * 16 + t
                for s in range(8):
                    csl = pl.ds(s * 16, 16)
                    buf[r, csl] = buf[r, csl] * sp
            return 0

        lax.fori_loop(0, K // 16, jbody, 0)
        pltpu.sync_copy(buf, res_sh.at[didx.at[b]], add=True)

    start(0, 0)

    def pair(p, _):
        c0 = 2 * p
        start(c0 + 1, 1)
        process(c0, 0)
        start(jnp.minimum(c0 + 2, NCH1 - 2), 0)
        process(c0 + 1, 1)
        return 0

    lax.fori_loop(0, NCH1 // 2, pair, 0)
    pltpu.make_async_copy(z_h.at[pl.ds(0, K)], rows0, sem0).wait()

    plsc.subcore_barrier()
    rows_per_tile = NP // NT1
    fbase = sid * rows_per_tile
    pltpu.sync_copy(res_sh.at[pl.ds(fbase, rows_per_tile)],
                    res_h.at[pl.ds(fbase, rows_per_tile)])


def _tc1_body(x_ref, w1_ref, w2_ref, ap_ref, z_ref, s_ref):
    z1 = jnp.maximum(
        lax.dot_general(x_ref[...], w1_ref[...], (((1,), (1,)), ((), ())),
                        preferred_element_type=jnp.float32), 0.0)
    z2 = jnp.maximum(
        lax.dot_general(z1, w2_ref[...], (((1,), (1,)), ((), ())),
                        preferred_element_type=jnp.float32), 0.0)
    z_ref[...] = z2
    s_ref[...] = lax.dot_general(z2, ap_ref[...], (((1,), (0,)), ((), ())),
                                 preferred_element_type=jnp.float32)


def _tc2_body(z_ref, r_ref, o_ref):
    o_ref[...] = jnp.maximum(z_ref[...] - r_ref[...], 0.0)


_B1 = 1024


def _dense_fwd(xp, W1, W2, ap):
    return pl.pallas_call(
        _tc1_body,
        grid=(NP // _B1,),
        in_specs=[
            pl.BlockSpec((_B1, D), lambda i: (i, 0)),
            pl.BlockSpec((D, D), lambda i: (0, 0)),
            pl.BlockSpec((D, D), lambda i: (0, 0)),
            pl.BlockSpec((D, D), lambda i: (0, 0)),
        ],
        out_specs=[
            pl.BlockSpec((_B1, D), lambda i: (i, 0)),
            pl.BlockSpec((_B1, D), lambda i: (i, 0)),
        ],
        out_shape=[
            jax.ShapeDtypeStruct((NP, D), jnp.float32),
            jax.ShapeDtypeStruct((NP, D), jnp.float32),
        ],
    )(xp, W1, W2, ap)


def _final(z, r):
    return pl.pallas_call(
        _tc2_body,
        grid=(NP // _B1,),
        in_specs=[
            pl.BlockSpec((_B1, D), lambda i: (i, 0)),
            pl.BlockSpec((_B1, D), lambda i: (i, 0)),
        ],
        out_specs=pl.BlockSpec((_B1, D), lambda i: (i, 0)),
        out_shape=jax.ShapeDtypeStruct((NP, D), jnp.float32),
    )(z, r)


def kernel(x, edge_index, W1, W2, a):
    xp = jnp.zeros((NP, D), jnp.float32).at[:N].set(x)
    ap = jnp.zeros((D, D), jnp.float32)
    ap = ap.at[:, 0].set(a[:D, 0]).at[:, 1].set(a[D:, 0])

    pad = jnp.full((EP - E,), N, jnp.int32)
    src = jnp.concatenate([edge_index[0], pad])
    dst = jnp.concatenate([edge_index[1], pad])
    src2 = src.reshape(NT, EPT)
    dst2 = dst.reshape(NT, EPT)
    src1 = src.reshape(NT1, EPT1)
    dst1 = dst.reshape(NT1, EPT1)

    z, sfull = _dense_fwd(xp, W1, W2, ap)
    s1 = sfull[:, 0]
    s2 = sfull[:, 1]

    mpart = _segmax(s1, s2, src2, dst2)
    dpart, eexp = _denom(s1, s2, src2, dst2, mpart)
    wts = _weights(dpart, eexp, dst2)
    res = _aggregate(z, src1, dst1, wts.reshape(NT1, EPT1))

    out = _final(z, res)
    return out[:N]


# speedup vs baseline: 13.0822x; 1.8373x over previous
import functools

import jax
import jax.numpy as jnp
from jax import lax
from jax.experimental import pallas as pl
from jax.experimental.pallas import tpu as pltpu
from jax.experimental.pallas import tpu_sc as plsc

N = 10000
E = 320000
D = 128
NP = 10240
EP = 327680
NT = 32
EPT = EP // NT
K = 128
NCH = EPT // K
VPT = EPT // 16
NV = NP // 16

_mesh = plsc.VectorSubcoreMesh(core_axis_name="c", subcore_axis_name="s")
_sc_params = pltpu.CompilerParams(needs_layout_passes=False)


def _lane():
    return lax.broadcasted_iota(jnp.int32, (16,), 0)


_GDN = lax.GatherDimensionNumbers(
    offset_dims=(), collapsed_slice_dims=(0,), start_index_map=(0,))


def _take16(v, idx):
    return lax.gather(v, idx[:, None], _GDN, (1,),
                      mode=lax.GatherScatterMode.PROMISE_IN_BOUNDS)


def _seg_combine(k, v, op):
    lane = _lane()
    for sh in (1, 2, 4, 8):
        idx = jnp.maximum(lane - sh, 0)
        kz = _take16(k, idx)
        vz = _take16(v, idx)
        v = jnp.where((kz == k) & (lane >= sh), op(v, vz), v)
    nxt = _take16(k, jnp.minimum(lane + 1, 15))
    last = (k != nxt) | (lane == 15)
    return v, last


def _edge_logit(s1_v, s2_v, d16, s16):
    e = plsc.load_gather(s1_v, [d16]) + plsc.load_gather(s2_v, [s16])
    return jnp.where(e >= 0, e, 0.01 * e)


def _wid():
    return lax.axis_index("c") * 16 + lax.axis_index("s")


SLW = NP // 16


def _combine_sliced(part_h, buf_v, slc_v, sh, acc_v, op, post=None):
    s = lax.axis_index("s")
    off = s * SLW
    pltpu.sync_copy(part_h.at[:, pl.ds(off, SLW)], buf_v)

    def vloop(j, _):
        sl = pl.ds(j * 16, 16)

        def rloop(r, acc):
            return op(acc, buf_v[r, sl])

        acc = lax.fori_loop(1, NT, rloop, buf_v[0, sl])
        slc_v[sl] = acc if post is None else post(acc)
        return 0

    lax.fori_loop(0, SLW // 16, vloop, 0)
    pltpu.sync_copy(slc_v, sh.at[pl.ds(off, SLW)])
    plsc.subcore_barrier()
    pltpu.sync_copy(sh, acc_v)


@functools.partial(
    pl.kernel,
    out_type=jax.ShapeDtypeStruct((NT, NP), jnp.float32),
    mesh=_mesh,
    compiler_params=_sc_params,
    scratch_types=[
        pltpu.VMEM((NP,), jnp.float32),
        pltpu.VMEM((NP,), jnp.float32),
        pltpu.VMEM((EPT,), jnp.int32),
        pltpu.VMEM((EPT,), jnp.int32),
        pltpu.VMEM((NP,), jnp.float32),
    ],
)
def _segmax(s1_h, s2_h, src_h, dst_h, mpart_h, s1_v, s2_v, src_v, dst_v, m_v):
    w = _wid()
    pltpu.sync_copy(s1_h, s1_v)
    pltpu.sync_copy(s2_h, s2_v)
    pltpu.sync_copy(src_h.at[w], src_v)
    pltpu.sync_copy(dst_h.at[w], dst_v)

    def init(j, _):
        m_v[pl.ds(j * 16, 16)] = jnp.full((16,), -1e30, jnp.float32)
        return 0

    lax.fori_loop(0, NV, init, 0)

    def step(j, _):
        sl = pl.ds(j * 16, 16)
        d16 = dst_v[sl]
        e = _edge_logit(s1_v, s2_v, d16, src_v[sl])
        k, v = plsc.sort_key_val(d16, e)
        v, last = _seg_combine(k, v, jnp.maximum)
        old = plsc.load_gather(m_v, [k])
        plsc.store_scatter(m_v, [k], jnp.maximum(old, v), mask=last)
        return 0

    lax.fori_loop(0, VPT, step, 0)
    pltpu.sync_copy(m_v, mpart_h.at[w])


@functools.partial(
    pl.kernel,
    out_type=(
        jax.ShapeDtypeStruct((NT, NP), jnp.float32),
        jax.ShapeDtypeStruct((NT, EPT), jnp.float32),
    ),
    mesh=_mesh,
    compiler_params=_sc_params,
    scratch_types=[
        pltpu.VMEM((NP,), jnp.float32),
        pltpu.VMEM((NP,), jnp.float32),
        pltpu.VMEM((EPT,), jnp.int32),
        pltpu.VMEM((EPT,), jnp.int32),
        pltpu.VMEM((NP,), jnp.float32),
        pltpu.VMEM((NT, SLW), jnp.float32),
        pltpu.VMEM((SLW,), jnp.float32),
        pltpu.VMEM_SHARED((NP,), jnp.float32),
        pltpu.VMEM((NP,), jnp.float32),
        pltpu.VMEM((EPT,), jnp.float32),
    ],
)
def _denom(s1_h, s2_h, src_h, dst_h, mpart_h, dpart_h, eexp_h,
           s1_v, s2_v, src_v, dst_v, m_v, buf_v, slc_v, m_sh, den_v, ex_v):
    w = _wid()
    pltpu.sync_copy(s1_h, s1_v)
    pltpu.sync_copy(s2_h, s2_v)
    pltpu.sync_copy(src_h.at[w], src_v)
    pltpu.sync_copy(dst_h.at[w], dst_v)
    _combine_sliced(mpart_h, buf_v, slc_v, m_sh, m_v, jnp.maximum)

    def init(j, _):
        den_v[pl.ds(j * 16, 16)] = jnp.zeros((16,), jnp.float32)
        return 0

    lax.fori_loop(0, NV, init, 0)

    def step(j, _):
        sl = pl.ds(j * 16, 16)
        d16 = dst_v[sl]
        e = _edge_logit(s1_v, s2_v, d16, src_v[sl])
        ex = jnp.exp(e - plsc.load_gather(m_v, [d16]))
        ex_v[sl] = ex
        k, v = plsc.sort_key_val(d16, ex)
        v, last = _seg_combine(k, v, lambda x, y: x + y)
        plsc.addupdate_scatter(den_v, [k], v, mask=last)
        return 0

    lax.fori_loop(0, VPT, step, 0)
    pltpu.sync_copy(den_v, dpart_h.at[w])
    pltpu.sync_copy(ex_v, eexp_h.at[w])


@functools.partial(
    pl.kernel,
    out_type=jax.ShapeDtypeStruct((NT, EPT), jnp.float32),
    mesh=_mesh,
    compiler_params=_sc_params,
    scratch_types=[
        pltpu.VMEM((EPT,), jnp.int32),
        pltpu.VMEM((EPT,), jnp.float32),
        pltpu.VMEM((NT, SLW), jnp.float32),
        pltpu.VMEM((SLW,), jnp.float32),
        pltpu.VMEM_SHARED((NP,), jnp.float32),
        pltpu.VMEM((NP,), jnp.float32),
    ],
)
def _weights(dpart_h, eexp_h, dst_h, w_h, dst_v, ex_v, buf_v, slc_v, i_sh,
             inv_v):
    w = _wid()
    pltpu.sync_copy(dst_h.at[w], dst_v)
    pltpu.sync_copy(eexp_h.at[w], ex_v)
    _combine_sliced(dpart_h, buf_v, slc_v, i_sh, inv_v,
                    lambda x, y: x + y, post=lambda v: 1.0 / v)

    def wconv(j, _):
        sl = pl.ds(j * 16, 16)
        ex_v[sl] = ex_v[sl] * plsc.load_gather(inv_v, [dst_v[sl]])
        return 0

    lax.fori_loop(0, VPT, wconv, 0)
    pltpu.sync_copy(ex_v, w_h.at[w])


@functools.partial(
    pl.kernel,
    out_type=jax.ShapeDtypeStruct((2, NP, D), jnp.float32),
    mesh=_mesh,
    compiler_params=_sc_params,
    scratch_types=[
        pltpu.VMEM((2, K), jnp.int32),
        pltpu.VMEM((2, K), jnp.int32),
        pltpu.VMEM((2, K), jnp.float32),
        pltpu.VMEM((K, D), jnp.float32),
        pltpu.VMEM((K, D), jnp.float32),
        pltpu.VMEM_SHARED((NP, D), jnp.float32),
        pltpu.SemaphoreType.DMA,
        pltpu.SemaphoreType.DMA,
    ],
)
def _aggregate(z_h, src_h, dst_h, w_h, res_h,
               sidx, didx, wbuf, rows0, rows1, res_sh, sem0, sem1):
    cid = lax.axis_index("c")
    sid = lax.axis_index("s")
    w = _wid()
    bufs = (rows0, rows1)
    sems = (sem0, sem1)

    def zrow(r, _):
        for s in range(8):
            rows0[r, pl.ds(s * 16, 16)] = jnp.zeros((16,), jnp.float32)
        return 0

    lax.fori_loop(0, K, zrow, 0)
    rows_per_tile = NP // 16
    base = sid * rows_per_tile
    for q in range(rows_per_tile // K):
        pltpu.sync_copy(rows0, res_sh.at[pl.ds(base + q * K, K)])
    plsc.subcore_barrier()

    def start(c, b):
        pltpu.sync_copy(src_h.at[w, pl.ds(c * K, K)], sidx.at[b])
        pltpu.async_copy(z_h.at[sidx.at[b]], bufs[b], sems[b])

    def process(c, b):
        pltpu.sync_copy(dst_h.at[w, pl.ds(c * K, K)], didx.at[b])
        pltpu.sync_copy(w_h.at[w, pl.ds(c * K, K)], wbuf.at[b])
        pltpu.make_async_copy(z_h.at[pl.ds(0, K)], bufs[b], sems[b]).wait()
        buf = bufs[b]

        def jbody(j, _):
            w16 = wbuf[b, pl.ds(j * 16, 16)]
            for t in range(16):
                sp = _take16(w16, jnp.full((16,), t, jnp.int32))
                r = j * 16 + t
                for s in range(8):
                    csl = pl.ds(s * 16, 16)
                    buf[r, csl] = buf[r, csl] * sp
            return 0

        lax.fori_loop(0, K // 16, jbody, 0)
        pltpu.sync_copy(buf, res_sh.at[didx.at[b]], add=True)

    start(0, 0)

    def pair(p, _):
        c0 = 2 * p
        start(c0 + 1, 1)
        process(c0, 0)
        start(jnp.minimum(c0 + 2, NCH - 2), 0)
        process(c0 + 1, 1)
        return 0

    lax.fori_loop(0, NCH // 2, pair, 0)
    pltpu.make_async_copy(z_h.at[pl.ds(0, K)], rows0, sem0).wait()

    plsc.subcore_barrier()
    fbase = sid * rows_per_tile
    pltpu.sync_copy(res_sh.at[pl.ds(fbase, rows_per_tile)],
                    res_h.at[cid, pl.ds(fbase, rows_per_tile)])


def _tc1_body(x_ref, w1_ref, w2_ref, ap_ref, z_ref, s_ref):
    z1 = jnp.maximum(
        lax.dot_general(x_ref[...], w1_ref[...], (((1,), (1,)), ((), ())),
                        preferred_element_type=jnp.float32), 0.0)
    z2 = jnp.maximum(
        lax.dot_general(z1, w2_ref[...], (((1,), (1,)), ((), ())),
                        preferred_element_type=jnp.float32), 0.0)
    z_ref[...] = z2
    s_ref[...] = lax.dot_general(z2, ap_ref[...], (((1,), (0,)), ((), ())),
                                 preferred_element_type=jnp.float32)


def _tc2_body(z_ref, r0_ref, r1_ref, o_ref):
    o_ref[...] = jnp.maximum(z_ref[...] - r0_ref[...] - r1_ref[...], 0.0)


_B1 = 1024


def _dense_fwd(xp, W1, W2, ap):
    return pl.pallas_call(
        _tc1_body,
        grid=(NP // _B1,),
        in_specs=[
            pl.BlockSpec((_B1, D), lambda i: (i, 0)),
            pl.BlockSpec((D, D), lambda i: (0, 0)),
            pl.BlockSpec((D, D), lambda i: (0, 0)),
            pl.BlockSpec((D, D), lambda i: (0, 0)),
        ],
        out_specs=[
            pl.BlockSpec((_B1, D), lambda i: (i, 0)),
            pl.BlockSpec((_B1, D), lambda i: (i, 0)),
        ],
        out_shape=[
            jax.ShapeDtypeStruct((NP, D), jnp.float32),
            jax.ShapeDtypeStruct((NP, D), jnp.float32),
        ],
    )(xp, W1, W2, ap)


def _final(z, r0, r1):
    return pl.pallas_call(
        _tc2_body,
        grid=(NP // _B1,),
        in_specs=[
            pl.BlockSpec((_B1, D), lambda i: (i, 0)),
            pl.BlockSpec((_B1, D), lambda i: (i, 0)),
            pl.BlockSpec((_B1, D), lambda i: (i, 0)),
        ],
        out_specs=pl.BlockSpec((_B1, D), lambda i: (i, 0)),
        out_shape=jax.ShapeDtypeStruct((NP, D), jnp.float32),
    )(z, r0, r1)


def kernel(x, edge_index, W1, W2, a):
    xp = jnp.zeros((NP, D), jnp.float32).at[:N].set(x)
    ap = jnp.zeros((D, D), jnp.float32)
    ap = ap.at[:, 0].set(a[:D, 0]).at[:, 1].set(a[D:, 0])

    pad = jnp.full((EP - E,), N, jnp.int32)
    src = jnp.concatenate([edge_index[0], pad])
    dst = jnp.concatenate([edge_index[1], pad])
    src2 = src.reshape(NT, EPT)
    dst2 = dst.reshape(NT, EPT)

    z, sfull = _dense_fwd(xp, W1, W2, ap)
    s1 = sfull[:, 0]
    s2 = sfull[:, 1]

    mpart = _segmax(s1, s2, src2, dst2)
    dpart, eexp = _denom(s1, s2, src2, dst2, mpart)
    wts = _weights(dpart, eexp, dst2)
    res = _aggregate(z, src2, dst2, wts)

    out = _final(z, res[0], res[1])
    return out[:N]


# speedup vs baseline: 13.4550x; 1.0285x over previous
import functools

import jax
import jax.numpy as jnp
from jax import lax
from jax.experimental import pallas as pl
from jax.experimental.pallas import tpu as pltpu
from jax.experimental.pallas import tpu_sc as plsc

N = 10000
E = 320000
D = 128
NP = 10240
EP = 327680
NT = 32
EPT = EP // NT
K = 128
NCH = EPT // K
VPT = EPT // 16
NV = NP // 16

_mesh = plsc.VectorSubcoreMesh(core_axis_name="c", subcore_axis_name="s")
_sc_params = pltpu.CompilerParams(needs_layout_passes=False)


def _lane():
    return lax.broadcasted_iota(jnp.int32, (16,), 0)


_GDN = lax.GatherDimensionNumbers(
    offset_dims=(), collapsed_slice_dims=(0,), start_index_map=(0,))


def _take16(v, idx):
    return lax.gather(v, idx[:, None], _GDN, (1,),
                      mode=lax.GatherScatterMode.PROMISE_IN_BOUNDS)


def _seg_combine(k, v, op):
    lane = _lane()
    for sh in (1, 2, 4, 8):
        idx = jnp.maximum(lane - sh, 0)
        kz = _take16(k, idx)
        vz = _take16(v, idx)
        v = jnp.where((kz == k) & (lane >= sh), op(v, vz), v)
    nxt = _take16(k, jnp.minimum(lane + 1, 15))
    last = (k != nxt) | (lane == 15)
    return v, last


def _edge_logit(s1_v, s2_v, d16, s16):
    e = plsc.load_gather(s1_v, [d16]) + plsc.load_gather(s2_v, [s16])
    return jnp.where(e >= 0, e, 0.01 * e)


def _wid():
    return lax.axis_index("c") * 16 + lax.axis_index("s")


SLW = NP // 16


def _combine_sliced(part_h, buf_v, slc_v, sh, acc_v, op, post=None):
    s = lax.axis_index("s")
    off = s * SLW
    pltpu.sync_copy(part_h.at[:, pl.ds(off, SLW)], buf_v)

    def vloop(j, _):
        sl = pl.ds(j * 16, 16)

        def rloop(r, acc):
            return op(acc, buf_v[r, sl])

        acc = lax.fori_loop(1, NT, rloop, buf_v[0, sl])
        slc_v[sl] = acc if post is None else post(acc)
        return 0

    lax.fori_loop(0, SLW // 16, vloop, 0)
    pltpu.sync_copy(slc_v, sh.at[pl.ds(off, SLW)])
    plsc.subcore_barrier()
    pltpu.sync_copy(sh, acc_v)


@functools.partial(
    pl.kernel,
    out_type=jax.ShapeDtypeStruct((NT, NP), jnp.float32),
    mesh=_mesh,
    compiler_params=_sc_params,
    scratch_types=[
        pltpu.VMEM((NP,), jnp.float32),
        pltpu.VMEM((NP,), jnp.float32),
        pltpu.VMEM((EPT,), jnp.int32),
        pltpu.VMEM((EPT,), jnp.int32),
        pltpu.VMEM((NP,), jnp.float32),
    ],
)
def _segmax(s1_h, s2_h, src_h, dst_h, mpart_h, s1_v, s2_v, src_v, dst_v, m_v):
    w = _wid()
    pltpu.sync_copy(s1_h, s1_v)
    pltpu.sync_copy(s2_h, s2_v)
    pltpu.sync_copy(src_h.at[w], src_v)
    pltpu.sync_copy(dst_h.at[w], dst_v)

    def init(j, _):
        m_v[pl.ds(j * 16, 16)] = jnp.full((16,), -1e30, jnp.float32)
        return 0

    lax.fori_loop(0, NV, init, 0)

    def step(j, _):
        sl = pl.ds(j * 16, 16)
        d16 = dst_v[sl]
        e = _edge_logit(s1_v, s2_v, d16, src_v[sl])
        k, v = plsc.sort_key_val(d16, e)
        v, last = _seg_combine(k, v, jnp.maximum)
        old = plsc.load_gather(m_v, [k])
        plsc.store_scatter(m_v, [k], jnp.maximum(old, v), mask=last)
        return 0

    lax.fori_loop(0, VPT, step, 0)
    pltpu.sync_copy(m_v, mpart_h.at[w])


@functools.partial(
    pl.kernel,
    out_type=(
        jax.ShapeDtypeStruct((NT, NP), jnp.float32),
        jax.ShapeDtypeStruct((NT, EPT), jnp.float32),
    ),
    mesh=_mesh,
    compiler_params=_sc_params,
    scratch_types=[
        pltpu.VMEM((NP,), jnp.float32),
        pltpu.VMEM((NP,), jnp.float32),
        pltpu.VMEM((EPT,), jnp.int32),
        pltpu.VMEM((EPT,), jnp.int32),
        pltpu.VMEM((NP,), jnp.float32),
        pltpu.VMEM((NT, SLW), jnp.float32),
        pltpu.VMEM((SLW,), jnp.float32),
        pltpu.VMEM_SHARED((NP,), jnp.float32),
        pltpu.VMEM((NP,), jnp.float32),
        pltpu.VMEM((EPT,), jnp.float32),
    ],
)
def _denom(s1_h, s2_h, src_h, dst_h, mpart_h, dpart_h, eexp_h,
           s1_v, s2_v, src_v, dst_v, m_v, buf_v, slc_v, m_sh, den_v, ex_v):
    w = _wid()
    pltpu.sync_copy(s1_h, s1_v)
    pltpu.sync_copy(s2_h, s2_v)
    pltpu.sync_copy(src_h.at[w], src_v)
    pltpu.sync_copy(dst_h.at[w], dst_v)
    _combine_sliced(mpart_h, buf_v, slc_v, m_sh, m_v, jnp.maximum)

    def init(j, _):
        den_v[pl.ds(j * 16, 16)] = jnp.zeros((16,), jnp.float32)
        return 0

    lax.fori_loop(0, NV, init, 0)

    def step(j, _):
        sl = pl.ds(j * 16, 16)
        d16 = dst_v[sl]
        e = _edge_logit(s1_v, s2_v, d16, src_v[sl])
        ex = jnp.exp(e - plsc.load_gather(m_v, [d16]))
        ex_v[sl] = ex
        k, v = plsc.sort_key_val(d16, ex)
        v, last = _seg_combine(k, v, lambda x, y: x + y)
        plsc.addupdate_scatter(den_v, [k], v, mask=last)
        return 0

    lax.fori_loop(0, VPT, step, 0)
    pltpu.sync_copy(den_v, dpart_h.at[w])
    pltpu.sync_copy(ex_v, eexp_h.at[w])


@functools.partial(
    pl.kernel,
    out_type=jax.ShapeDtypeStruct((2, NP, D), jnp.float32),
    mesh=_mesh,
    compiler_params=_sc_params,
    scratch_types=[
        pltpu.VMEM((2, K), jnp.int32),
        pltpu.VMEM((2, K), jnp.int32),
        pltpu.VMEM((2, K), jnp.float32),
        pltpu.VMEM((K, D), jnp.float32),
        pltpu.VMEM((K, D), jnp.float32),
        pltpu.VMEM_SHARED((NP, D), jnp.float32),
        pltpu.SemaphoreType.DMA,
        pltpu.SemaphoreType.DMA,
    ],
)
def _aggregate(z_h, src_h, dst_h, w_h, res_h,
               sidx, didx, wbuf, rows0, rows1, res_sh, sem0, sem1):
    cid = lax.axis_index("c")
    sid = lax.axis_index("s")
    w = _wid()
    bufs = (rows0, rows1)
    sems = (sem0, sem1)

    def zrow(r, _):
        for s in range(8):
            rows0[r, pl.ds(s * 16, 16)] = jnp.zeros((16,), jnp.float32)
        return 0

    lax.fori_loop(0, K, zrow, 0)
    rows_per_tile = NP // 16
    base = sid * rows_per_tile
    for q in range(rows_per_tile // K):
        pltpu.sync_copy(rows0, res_sh.at[pl.ds(base + q * K, K)])
    plsc.subcore_barrier()

    def start(c, b):
        pltpu.sync_copy(src_h.at[w, pl.ds(c * K, K)], sidx.at[b])
        pltpu.async_copy(z_h.at[sidx.at[b]], bufs[b], sems[b])

    def process(c, b):
        pltpu.sync_copy(dst_h.at[w, pl.ds(c * K, K)], didx.at[b])
        pltpu.sync_copy(w_h.at[w, pl.ds(c * K, K)], wbuf.at[b])
        pltpu.make_async_copy(z_h.at[pl.ds(0, K)], bufs[b], sems[b]).wait()
        buf = bufs[b]

        def jbody(j, _):
            w16 = wbuf[b, pl.ds(j * 16, 16)]
            for t in range(16):
                sp = _take16(w16, jnp.full((16,), t, jnp.int32))
                r = j * 16 + t
                for s in range(8):
                    csl = pl.ds(s * 16, 16)
                    buf[r, csl] = buf[r, csl] * sp
            return 0

        lax.fori_loop(0, K // 16, jbody, 0)
        pltpu.sync_copy(buf, res_sh.at[didx.at[b]], add=True)

    start(0, 0)

    def pair(p, _):
        c0 = 2 * p
        start(c0 + 1, 1)
        process(c0, 0)
        start(jnp.minimum(c0 + 2, NCH - 2), 0)
        process(c0 + 1, 1)
        return 0

    lax.fori_loop(0, NCH // 2, pair, 0)
    pltpu.make_async_copy(z_h.at[pl.ds(0, K)], rows0, sem0).wait()

    plsc.subcore_barrier()
    fbase = sid * rows_per_tile
    pltpu.sync_copy(res_sh.at[pl.ds(fbase, rows_per_tile)],
                    res_h.at[cid, pl.ds(fbase, rows_per_tile)])


def _tc1_body(x_ref, w1_ref, w2_ref, ap_ref, z_ref, s_ref):
    z1 = jnp.maximum(
        lax.dot_general(x_ref[...], w1_ref[...], (((1,), (1,)), ((), ())),
                        preferred_element_type=jnp.float32), 0.0)
    z2 = jnp.maximum(
        lax.dot_general(z1, w2_ref[...], (((1,), (1,)), ((), ())),
                        preferred_element_type=jnp.float32), 0.0)
    z_ref[...] = z2
    s_ref[...] = lax.dot_general(z2, ap_ref[...], (((1,), (0,)), ((), ())),
                                 preferred_element_type=jnp.float32)


def _tc2_body(z_ref, r0_ref, r1_ref, dpart_ref, o_ref):
    den = jnp.sum(dpart_ref[...], axis=0)
    den = jnp.where(den == 0.0, 1.0, den)
    res = (r0_ref[...] + r1_ref[...]) * (1.0 / den)[:, None]
    o_ref[...] = jnp.maximum(z_ref[...] - res, 0.0)


_B1 = 1024


def _dense_fwd(xp, W1, W2, ap):
    return pl.pallas_call(
        _tc1_body,
        grid=(NP // _B1,),
        in_specs=[
            pl.BlockSpec((_B1, D), lambda i: (i, 0)),
            pl.BlockSpec((D, D), lambda i: (0, 0)),
            pl.BlockSpec((D, D), lambda i: (0, 0)),
            pl.BlockSpec((D, D), lambda i: (0, 0)),
        ],
        out_specs=[
            pl.BlockSpec((_B1, D), lambda i: (i, 0)),
            pl.BlockSpec((_B1, D), lambda i: (i, 0)),
        ],
        out_shape=[
            jax.ShapeDtypeStruct((NP, D), jnp.float32),
            jax.ShapeDtypeStruct((NP, D), jnp.float32),
        ],
    )(xp, W1, W2, ap)


def _final(z, r0, r1, dpart):
    return pl.pallas_call(
        _tc2_body,
        grid=(NP // _B1,),
        in_specs=[
            pl.BlockSpec((_B1, D), lambda i: (i, 0)),
            pl.BlockSpec((_B1, D), lambda i: (i, 0)),
            pl.BlockSpec((_B1, D), lambda i: (i, 0)),
            pl.BlockSpec((NT, _B1), lambda i: (0, i)),
        ],
        out_specs=pl.BlockSpec((_B1, D), lambda i: (i, 0)),
        out_shape=jax.ShapeDtypeStruct((NP, D), jnp.float32),
    )(z, r0, r1, dpart)


def kernel(x, edge_index, W1, W2, a):
    xp = jnp.zeros((NP, D), jnp.float32).at[:N].set(x)
    ap = jnp.zeros((D, D), jnp.float32)
    ap = ap.at[:, 0].set(a[:D, 0]).at[:, 1].set(a[D:, 0])

    pad_src = jnp.full((EP - E,), N, jnp.int32)
    pad_dst = N + jnp.arange(EP - E, dtype=jnp.int32) % (NP - N)
    src = jnp.concatenate([edge_index[0], pad_src])
    dst = jnp.concatenate([edge_index[1], pad_dst])
    src2 = src.reshape(NT, EPT)
    dst2 = dst.reshape(NT, EPT)

    z, sfull = _dense_fwd(xp, W1, W2, ap)
    s1 = sfull[:, 0]
    s2 = sfull[:, 1]

    mpart = _segmax(s1, s2, src2, dst2)
    dpart, eexp = _denom(s1, s2, src2, dst2, mpart)
    res = _aggregate(z, src2, dst2, eexp)

    out = _final(z, res[0], res[1], dpart)
    return out[:N]


# speedup vs baseline: 15.2929x; 1.1366x over previous
import functools

import jax
import jax.numpy as jnp
from jax import lax
from jax.experimental import pallas as pl
from jax.experimental.pallas import tpu as pltpu
from jax.experimental.pallas import tpu_sc as plsc

N = 10000
E = 320000
D = 128
NP = 10240
NT = 32
K = 112
NCH = 90
EPT = NCH * K
EP = NT * EPT
VPT = EPT // 16
NV = NP // 16
NRES = NP
RPT = NRES // 16

_mesh = plsc.VectorSubcoreMesh(core_axis_name="c", subcore_axis_name="s")
_sc_params = pltpu.CompilerParams(needs_layout_passes=False)


def _lane():
    return lax.broadcasted_iota(jnp.int32, (16,), 0)


_GDN = lax.GatherDimensionNumbers(
    offset_dims=(), collapsed_slice_dims=(0,), start_index_map=(0,))


def _take16(v, idx):
    return lax.gather(v, idx[:, None], _GDN, (1,),
                      mode=lax.GatherScatterMode.PROMISE_IN_BOUNDS)


def _seg_combine(k, v, op):
    lane = _lane()
    for sh in (1, 2, 4, 8):
        idx = jnp.maximum(lane - sh, 0)
        kz = _take16(k, idx)
        vz = _take16(v, idx)
        v = jnp.where((kz == k) & (lane >= sh), op(v, vz), v)
    nxt = _take16(k, jnp.minimum(lane + 1, 15))
    last = (k != nxt) | (lane == 15)
    return v, last


def _edge_logit(s1_v, s2_v, d16, s16):
    e = plsc.load_gather(s1_v, [d16]) + plsc.load_gather(s2_v, [s16])
    return jnp.where(e >= 0, e, 0.01 * e)


def _wid():
    return lax.axis_index("c") * 16 + lax.axis_index("s")


SLW = NP // 16


def _combine_sliced(part_h, buf_v, slc_v, sh, acc_v, op, post=None):
    s = lax.axis_index("s")
    off = s * SLW
    pltpu.sync_copy(part_h.at[:, pl.ds(off, SLW)], buf_v)

    def vloop(j, _):
        sl = pl.ds(j * 16, 16)

        def rloop(r, acc):
            return op(acc, buf_v[r, sl])

        acc = lax.fori_loop(1, NT, rloop, buf_v[0, sl])
        slc_v[sl] = acc if post is None else post(acc)
        return 0

    lax.fori_loop(0, SLW // 16, vloop, 0)
    pltpu.sync_copy(slc_v, sh.at[pl.ds(off, SLW)])
    plsc.subcore_barrier()
    pltpu.sync_copy(sh, acc_v)


@functools.partial(
    pl.kernel,
    out_type=jax.ShapeDtypeStruct((NT, NP), jnp.float32),
    mesh=_mesh,
    compiler_params=_sc_params,
    scratch_types=[
        pltpu.VMEM((NP,), jnp.float32),
        pltpu.VMEM((NP,), jnp.float32),
        pltpu.VMEM((EPT,), jnp.int32),
        pltpu.VMEM((EPT,), jnp.int32),
        pltpu.VMEM((NP,), jnp.float32),
    ],
)
def _segmax(s1_h, s2_h, src_h, dst_h, mpart_h, s1_v, s2_v, src_v, dst_v, m_v):
    w = _wid()
    pltpu.sync_copy(s1_h, s1_v)
    pltpu.sync_copy(s2_h, s2_v)
    pltpu.sync_copy(src_h.at[w], src_v)
    pltpu.sync_copy(dst_h.at[w], dst_v)

    def init(j, _):
        m_v[pl.ds(j * 16, 16)] = jnp.full((16,), -1e30, jnp.float32)
        return 0

    lax.fori_loop(0, NV, init, 0)

    def step(j, _):
        sl = pl.ds(j * 16, 16)
        d16 = dst_v[sl]
        e = _edge_logit(s1_v, s2_v, d16, src_v[sl])
        k, v = plsc.sort_key_val(d16, e)
        v, last = _seg_combine(k, v, jnp.maximum)
        old = plsc.load_gather(m_v, [k])
        plsc.store_scatter(m_v, [k], jnp.maximum(old, v), mask=last)
        return 0

    lax.fori_loop(0, VPT, step, 0)
    pltpu.sync_copy(m_v, mpart_h.at[w])


@functools.partial(
    pl.kernel,
    out_type=(
        jax.ShapeDtypeStruct((NT, NP), jnp.float32),
        jax.ShapeDtypeStruct((NT, EPT), jnp.float32),
    ),
    mesh=_mesh,
    compiler_params=_sc_params,
    scratch_types=[
        pltpu.VMEM((NP,), jnp.float32),
        pltpu.VMEM((NP,), jnp.float32),
        pltpu.VMEM((EPT,), jnp.int32),
        pltpu.VMEM((EPT,), jnp.int32),
        pltpu.VMEM((NP,), jnp.float32),
        pltpu.VMEM((NT, SLW), jnp.float32),
        pltpu.VMEM((SLW,), jnp.float32),
        pltpu.VMEM_SHARED((NP,), jnp.float32),
        pltpu.VMEM((NP,), jnp.float32),
        pltpu.VMEM((EPT,), jnp.float32),
    ],
)
def _denom(s1_h, s2_h, src_h, dst_h, mpart_h, dpart_h, eexp_h,
           s1_v, s2_v, src_v, dst_v, m_v, buf_v, slc_v, m_sh, den_v, ex_v):
    w = _wid()
    pltpu.sync_copy(s1_h, s1_v)
    pltpu.sync_copy(s2_h, s2_v)
    pltpu.sync_copy(src_h.at[w], src_v)
    pltpu.sync_copy(dst_h.at[w], dst_v)
    _combine_sliced(mpart_h, buf_v, slc_v, m_sh, m_v, jnp.maximum)

    def init(j, _):
        den_v[pl.ds(j * 16, 16)] = jnp.zeros((16,), jnp.float32)
        return 0

    lax.fori_loop(0, NV, init, 0)

    def step(j, _):
        sl = pl.ds(j * 16, 16)
        d16 = dst_v[sl]
        e = _edge_logit(s1_v, s2_v, d16, src_v[sl])
        ex = jnp.exp(e - plsc.load_gather(m_v, [d16]))
        ex_v[sl] = ex
        k, v = plsc.sort_key_val(d16, ex)
        v, last = _seg_combine(k, v, lambda x, y: x + y)
        plsc.addupdate_scatter(den_v, [k], v, mask=last)
        return 0

    lax.fori_loop(0, VPT, step, 0)
    pltpu.sync_copy(den_v, dpart_h.at[w])
    pltpu.sync_copy(ex_v, eexp_h.at[w])


@functools.partial(
    pl.kernel,
    out_type=jax.ShapeDtypeStruct((2, NP, D), jnp.float32),
    mesh=_mesh,
    compiler_params=_sc_params,
    scratch_types=[
        pltpu.VMEM((3, K), jnp.int32),
        pltpu.VMEM((3, K), jnp.int32),
        pltpu.VMEM((3, K), jnp.float32),
        pltpu.VMEM((K, D), jnp.float32),
        pltpu.VMEM((K, D), jnp.float32),
        pltpu.VMEM((K, D), jnp.float32),
        pltpu.VMEM_SHARED((NRES, D), jnp.float32),
        pltpu.SemaphoreType.DMA,
        pltpu.SemaphoreType.DMA,
        pltpu.SemaphoreType.DMA,
        pltpu.SemaphoreType.DMA,
        pltpu.SemaphoreType.DMA,
        pltpu.SemaphoreType.DMA,
    ],
)
def _aggregate(z_h, src_h, dst_h, w_h, res_h,
               sidx, didx, wbuf, rows0, rows1, rows2, res_sh,
               g0, g1, g2, a0, a1, a2):
    cid = lax.axis_index("c")
    sid = lax.axis_index("s")
    w = _wid()
    bufs = (rows0, rows1, rows2)
    gsems = (g0, g1, g2)
    asems = (a0, a1, a2)

    def zrow(r, _):
        for s in range(8):
            rows0[r, pl.ds(s * 16, 16)] = jnp.zeros((16,), jnp.float32)
        return 0

    lax.fori_loop(0, K, zrow, 0)
    base = sid * RPT
    for q in range(RPT // K):
        pltpu.sync_copy(rows0, res_sh.at[pl.ds(base + q * K, K)])
    pltpu.sync_copy(rows0.at[pl.ds(0, RPT % K)],
                    res_sh.at[pl.ds(base + (RPT // K) * K, RPT % K)])
    plsc.subcore_barrier()

    ebase = w * EPT

    def gather(c, b):
        pltpu.sync_copy(src_h.at[pl.ds(ebase + c * K, K)], sidx.at[b])
        pltpu.async_copy(z_h.at[sidx.at[b]], bufs[b], gsems[b])

    def wait_gather(b):
        pltpu.make_async_copy(z_h.at[pl.ds(0, K)], bufs[b], gsems[b]).wait()

    def wait_sadd(b):
        pltpu.make_async_copy(bufs[b], res_sh.at[pl.ds(0, K)],
                              asems[b]).wait()

    def process(c, b):
        pltpu.sync_copy(dst_h.at[pl.ds(ebase + c * K, K)], didx.at[b])
        pltpu.sync_copy(w_h.at[pl.ds(ebase + c * K, K)], wbuf.at[b])
        wait_gather(b)
        buf = bufs[b]

        def jbody(j, _):
            w16 = wbuf[b, pl.ds(j * 16, 16)]
            for t in range(16):
                sp = _take16(w16, jnp.full((16,), t, jnp.int32))
                r = j * 16 + t
                for s in range(8):
                    csl = pl.ds(s * 16, 16)
                    buf[r, csl] = buf[r, csl] * sp
            return 0

        lax.fori_loop(0, K // 16, jbody, 0)
        pltpu.async_copy(buf, res_sh.at[didx.at[b]], asems[b], add=True)

    gather(0, 0)
    gather(1, 1)
    gather(2, 2)
    process(0, 0)
    wait_sadd(0)
    gather(3, 0)
    process(1, 1)
    wait_sadd(1)
    gather(4, 1)
    process(2, 2)

    def triple(t, _):
        c0 = 3 * t
        wait_sadd(2)
        gather(c0 + 2, 2)
        process(c0, 0)
        wait_sadd(0)
        gather(jnp.minimum(c0 + 3, NCH - 1), 0)
        process(c0 + 1, 1)
        wait_sadd(1)
        gather(jnp.minimum(c0 + 4, NCH - 1), 1)
        process(c0 + 2, 2)
        return 0

    lax.fori_loop(1, NCH // 3, triple, 0)

    wait_gather(0)
    wait_gather(1)
    wait_sadd(2)

    plsc.subcore_barrier()
    fbase = sid * RPT
    pltpu.sync_copy(res_sh.at[pl.ds(fbase, RPT)],
                    res_h.at[cid, pl.ds(fbase, RPT)])


def _tc1_body(x_ref, w1_ref, w2_ref, ap_ref, z_ref, s_ref):
    z1 = jnp.maximum(
        lax.dot_general(x_ref[...], w1_ref[...], (((1,), (1,)), ((), ())),
                        preferred_element_type=jnp.float32), 0.0)
    z2 = jnp.maximum(
        lax.dot_general(z1, w2_ref[...], (((1,), (1,)), ((), ())),
                        preferred_element_type=jnp.float32), 0.0)
    z_ref[...] = z2
    s_ref[...] = lax.dot_general(z2, ap_ref[...], (((1,), (0,)), ((), ())),
                                 preferred_element_type=jnp.float32)


def _tc2_body(z_ref, r0_ref, r1_ref, dpart_ref, o_ref):
    den = jnp.sum(dpart_ref[...], axis=0)
    den = jnp.where(den == 0.0, 1.0, den)
    res = (r0_ref[...] + r1_ref[...]) * (1.0 / den)[:, None]
    o_ref[...] = jnp.maximum(z_ref[...] - res, 0.0)


_B1 = 1024


def _dense_fwd(xp, W1, W2, ap):
    return pl.pallas_call(
        _tc1_body,
        grid=(NP // _B1,),
        in_specs=[
            pl.BlockSpec((_B1, D), lambda i: (i, 0)),
            pl.BlockSpec((D, D), lambda i: (0, 0)),
            pl.BlockSpec((D, D), lambda i: (0, 0)),
            pl.BlockSpec((D, D), lambda i: (0, 0)),
        ],
        out_specs=[
            pl.BlockSpec((_B1, D), lambda i: (i, 0)),
            pl.BlockSpec((_B1, D), lambda i: (i, 0)),
        ],
        out_shape=[
            jax.ShapeDtypeStruct((NP, D), jnp.float32),
            jax.ShapeDtypeStruct((NP, D), jnp.float32),
        ],
    )(xp, W1, W2, ap)


def _final(z, r0, r1, dpart):
    return pl.pallas_call(
        _tc2_body,
        grid=(NP // _B1,),
        in_specs=[
            pl.BlockSpec((_B1, D), lambda i: (i, 0)),
            pl.BlockSpec((_B1, D), lambda i: (i, 0)),
            pl.BlockSpec((_B1, D), lambda i: (i, 0)),
            pl.BlockSpec((NT, _B1), lambda i: (0, i)),
        ],
        out_specs=pl.BlockSpec((_B1, D), lambda i: (i, 0)),
        out_shape=jax.ShapeDtypeStruct((NP, D), jnp.float32),
    )(z, r0, r1, dpart)


def kernel(x, edge_index, W1, W2, a):
    xp = jnp.zeros((NP, D), jnp.float32).at[:N].set(x)
    ap = jnp.zeros((D, D), jnp.float32)
    ap = ap.at[:, 0].set(a[:D, 0]).at[:, 1].set(a[D:, 0])

    pad_src = jnp.full((EP - E,), N, jnp.int32)
    pad_dst = N + jnp.arange(EP - E, dtype=jnp.int32) % (NRES - N)
    src = jnp.concatenate([edge_index[0], pad_src])
    dst = jnp.concatenate([edge_index[1], pad_dst])
    src2 = src.reshape(NT, EPT)
    dst2 = dst.reshape(NT, EPT)

    z, sfull = _dense_fwd(xp, W1, W2, ap)
    s1 = sfull[:, 0]
    s2 = sfull[:, 1]

    mpart = _segmax(s1, s2, src2, dst2)
    dpart, eexp = _denom(s1, s2, src2, dst2, mpart)
    res = _aggregate(z, src, dst, eexp.reshape(EP))

    out = _final(z, res[0], res[1], dpart)
    return out[:N]
